# Initial kernel scaffold; baseline (speedup 1.0000x reference)
import functools

import numpy as np
import jax
import jax.numpy as jnp
from jax import lax
from jax.experimental import pallas as pl
from jax.experimental.pallas import tpu as pltpu
from jax.experimental.pallas import tpu_sc as plsc

B, T = 16384, 50
EMBED = 64
S = T + 2
LANES = 16
EV = EMBED // LANES

NC, NS = 2, 16
NW = NC * NS
ROWS_PER_W = B // NW
CB = 16
NCH = ROWS_PER_W // CB


def _positional_encoding(max_len, d_model):
    pos = np.arange(max_len, dtype=np.float32)[:, None]
    div = np.exp(np.arange(0, d_model, 2, dtype=np.float32) * (-np.log(10000.0) / d_model))
    pe = np.zeros((max_len, d_model), dtype=np.float32)
    pe[:, 0::2] = np.sin(pos * div)
    pe[:, 1::2] = np.cos(pos * div)
    return pe


_PE = jnp.asarray(_positional_encoding(T, EMBED))

_mesh = plsc.VectorSubcoreMesh(core_axis_name="c", subcore_axis_name="s")


@functools.partial(
    pl.kernel,
    mesh=_mesh,
    out_type=jax.ShapeDtypeStruct((B, S, EMBED), jnp.float32),
    scratch_types=[
        pltpu.VMEM((T, EMBED), jnp.float32),
        pltpu.VMEM((CB, T), jnp.int32),
        pltpu.VMEM((CB,), jnp.int32),
        pltpu.VMEM((CB,), jnp.int32),
        pltpu.VMEM((CB, T), jnp.float32),
        pltpu.VMEM((CB, EMBED), jnp.float32),
        pltpu.VMEM((CB, EMBED), jnp.float32),
        pltpu.VMEM((CB, S, EMBED), jnp.float32),
        pltpu.SemaphoreType.DMA,
    ],
)
def _sc_embed(uid_hbm, hist_hbm, tgt_hbm, rat_hbm, pe_hbm, tok_hbm, usr_hbm,
              out_hbm, pe_v, idx_v, uidx_v, tidx_v, rat_v, urows_v, trows_v,
              ob_v, sem):
    wid = lax.axis_index("s") * NC + lax.axis_index("c")
    pltpu.sync_copy(pe_hbm, pe_v)

    def chunk_body(ci, carry):
        b0 = wid * ROWS_PER_W + ci * CB
        pltpu.sync_copy(hist_hbm.at[pl.ds(b0, CB)], idx_v)
        pltpu.sync_copy(uid_hbm.at[pl.ds(b0, CB)], uidx_v)
        pltpu.sync_copy(tgt_hbm.at[pl.ds(b0, CB)], tidx_v)
        pltpu.sync_copy(rat_hbm.at[pl.ds(b0, CB)], rat_v)

        handles = []
        for b in range(CB):
            handles.append(
                pltpu.async_copy(tok_hbm.at[idx_v.at[b]],
                                 ob_v.at[b, pl.ds(1, T)], sem))
        handles.append(pltpu.async_copy(usr_hbm.at[uidx_v], urows_v, sem))
        handles.append(pltpu.async_copy(tok_hbm.at[tidx_v], trows_v, sem))
        for h in handles:
            h.wait()

        def t_body(t, c):
            for b in range(CB):
                r = rat_v[b, t]
                for e in range(EV):
                    sl = pl.ds(e * LANES, LANES)
                    ob_v[b, 1 + t, sl] = (ob_v[b, 1 + t, sl] + pe_v[t, sl]) * r
            return c

        lax.fori_loop(0, T, t_body, 0)

        for b in range(CB):
            for e in range(EV):
                sl = pl.ds(e * LANES, LANES)
                ob_v[b, 0, sl] = urows_v[b, sl]
                ob_v[b, S - 1, sl] = trows_v[b, sl]

        pltpu.sync_copy(ob_v, out_hbm.at[pl.ds(b0, CB)])
        return carry

    lax.fori_loop(0, NCH, chunk_body, 0)


def kernel(user_id, product_history, target_product_id, product_history_ratings,
           token_table, user_table):
    return _sc_embed(user_id, product_history, target_product_id,
                     product_history_ratings, _PE, token_table, user_table)


# speedup vs baseline: 1.4249x; 1.4249x over previous
import functools

import numpy as np
import jax
import jax.numpy as jnp
from jax import lax
from jax.experimental import pallas as pl
from jax.experimental.pallas import tpu as pltpu
from jax.experimental.pallas import tpu_sc as plsc

B, T = 16384, 50
EMBED = 64
S = T + 2
LANES = 16
EV = EMBED // LANES

NC, NS = 2, 16
NW = NC * NS
ROWS_PER_W = B // NW
CB = 16
TP = 56
NCH = ROWS_PER_W // CB


def _positional_encoding(max_len, d_model):
    pos = np.arange(max_len, dtype=np.float32)[:, None]
    div = np.exp(np.arange(0, d_model, 2, dtype=np.float32) * (-np.log(10000.0) / d_model))
    pe = np.zeros((max_len, d_model), dtype=np.float32)
    pe[:, 0::2] = np.sin(pos * div)
    pe[:, 1::2] = np.cos(pos * div)
    return pe


_PE = _positional_encoding(T, EMBED)

_mesh = plsc.VectorSubcoreMesh(core_axis_name="c", subcore_axis_name="s")

_GATHER_DNUMS = lax.GatherDimensionNumbers(
    offset_dims=(), collapsed_slice_dims=(0,), start_index_map=(0,))


@functools.partial(
    pl.kernel,
    mesh=_mesh,
    compiler_params=pltpu.CompilerParams(use_tc_tiling_on_sc=False),
    out_type=jax.ShapeDtypeStruct((B, S, EMBED), jnp.float32),
    scratch_types=[
        pltpu.VMEM((T * EMBED,), jnp.float32),
        pltpu.VMEM((CB * TP,), jnp.int32),
        pltpu.VMEM((CB,), jnp.int32),
        pltpu.VMEM((CB,), jnp.int32),
        pltpu.VMEM((CB * EMBED,), jnp.float32),
        pltpu.VMEM((CB, EMBED), jnp.float32),
        pltpu.VMEM((CB, EMBED), jnp.float32),
        pltpu.VMEM((CB, S, EMBED), jnp.float32),
        pltpu.SemaphoreType.DMA,
    ],
)
def _sc_embed(uid_hbm, hist_hbm, tgt_hbm, rat_hbm, pe_hbm, tok_hbm, usr_hbm,
              out_hbm, pe_v, idx_v, uidx_v, tidx_v, rat_v, urows_v, trows_v,
              ob_v, sem):
    wid = lax.axis_index("s") * NC + lax.axis_index("c")
    pltpu.sync_copy(pe_hbm, pe_v)

    def chunk_body(ci, carry):
        b0 = wid * ROWS_PER_W + ci * CB
        pltpu.sync_copy(hist_hbm.at[pl.ds(b0 * TP, CB * TP)], idx_v)
        pltpu.sync_copy(uid_hbm.at[pl.ds(b0, CB)], uidx_v)
        pltpu.sync_copy(tgt_hbm.at[pl.ds(b0, CB)], tidx_v)
        pltpu.sync_copy(rat_hbm.at[pl.ds(b0 * EMBED, CB * EMBED)], rat_v)

        handles = []
        for b in range(CB):
            handles.append(
                pltpu.async_copy(tok_hbm.at[idx_v.at[pl.ds(b * TP, T)]],
                                 ob_v.at[b, pl.ds(1, T)], sem))
        handles.append(pltpu.async_copy(usr_hbm.at[uidx_v], urows_v, sem))
        handles.append(pltpu.async_copy(tok_hbm.at[tidx_v], trows_v, sem))
        for h in handles:
            h.wait()

        def pair_body(bi, c):
            bs = [2 * bi, 2 * bi + 1]
            for tb in range((T + LANES - 1) // LANES):
                rvs = [rat_v[pl.ds(b * EMBED + tb * LANES, LANES)] for b in bs]
                for tt in range(min(LANES, T - tb * LANES)):
                    t = tb * LANES + tt
                    lane = jnp.full((LANES, 1), tt, dtype=jnp.int32)
                    pes = [pe_v[pl.ds(t * EMBED + e * LANES, LANES)]
                           for e in range(EV)]
                    for b, rv in zip(bs, rvs):
                        r = lax.gather(
                            rv, lane, _GATHER_DNUMS, (1,),
                            mode=lax.GatherScatterMode.PROMISE_IN_BOUNDS)
                        for e in range(EV):
                            sl = pl.ds(e * LANES, LANES)
                            ob_v[b, 1 + t, sl] = (ob_v[b, 1 + t, sl] + pes[e]) * r
            return c

        lax.fori_loop(0, CB // 2, pair_body, 0)

        for b in range(CB):
            for e in range(EV):
                sl = pl.ds(e * LANES, LANES)
                ob_v[b, 0, sl] = urows_v[b, sl]
                ob_v[b, S - 1, sl] = trows_v[b, sl]

        pltpu.sync_copy(ob_v, out_hbm.at[pl.ds(b0, CB)])
        return carry

    lax.fori_loop(0, NCH, chunk_body, 0)


def kernel(user_id, product_history, target_product_id, product_history_ratings,
           token_table, user_table):
    pe = jnp.asarray(_PE).reshape(-1)
    hist_p = jnp.pad(product_history, ((0, 0), (0, TP - T))).reshape(-1)
    rat_p = jnp.pad(product_history_ratings,
                    ((0, 0), (0, EMBED - T))).reshape(-1)
    return _sc_embed(user_id, hist_p, target_product_id, rat_p, pe,
                     token_table, user_table)


# speedup vs baseline: 1.5714x; 1.1029x over previous
import functools

import numpy as np
import jax
import jax.numpy as jnp
from jax import lax
from jax.experimental import pallas as pl
from jax.experimental.pallas import tpu as pltpu
from jax.experimental.pallas import tpu_sc as plsc

B, T = 16384, 50
EMBED = 64
S = T + 2
LANES = 16
EV = EMBED // LANES
TP = 56

NC, NS = 2, 16
NW = NC * NS
ROWS_PER_W = B // NW
CB = 16
NCH = ROWS_PER_W // CB


def _positional_encoding(max_len, d_model):
    pos = np.arange(max_len, dtype=np.float32)[:, None]
    div = np.exp(np.arange(0, d_model, 2, dtype=np.float32) * (-np.log(10000.0) / d_model))
    pe = np.zeros((max_len, d_model), dtype=np.float32)
    pe[:, 0::2] = np.sin(pos * div)
    pe[:, 1::2] = np.cos(pos * div)
    return pe


_PE = _positional_encoding(T, EMBED)

_mesh = plsc.VectorSubcoreMesh(core_axis_name="c", subcore_axis_name="s")

_GATHER_DNUMS = lax.GatherDimensionNumbers(
    offset_dims=(), collapsed_slice_dims=(0,), start_index_map=(0,))

_SLOT_SCRATCH = [
    pltpu.VMEM((CB * TP,), jnp.int32),
    pltpu.VMEM((CB,), jnp.int32),
    pltpu.VMEM((CB,), jnp.int32),
    pltpu.VMEM((CB * EMBED,), jnp.float32),
    pltpu.VMEM((CB, EMBED), jnp.float32),
    pltpu.VMEM((CB, EMBED), jnp.float32),
    pltpu.VMEM((CB, S, EMBED), jnp.float32),
]


@functools.partial(
    pl.kernel,
    mesh=_mesh,
    compiler_params=pltpu.CompilerParams(use_tc_tiling_on_sc=False),
    out_type=jax.ShapeDtypeStruct((B, S, EMBED), jnp.float32),
    scratch_types=[
        pltpu.VMEM((T * EMBED,), jnp.float32),
        *_SLOT_SCRATCH,
        *_SLOT_SCRATCH,
        pltpu.SemaphoreType.DMA,
        pltpu.SemaphoreType.DMA,
        pltpu.SemaphoreType.DMA,
        pltpu.SemaphoreType.DMA,
        pltpu.SemaphoreType.DMA,
    ],
)
def _sc_embed(uid_hbm, hist_hbm, tgt_hbm, rat_hbm, pe_hbm, tok_hbm, usr_hbm,
              out_hbm, pe_v,
              idx0, uidx0, tidx0, rat0, urows0, trows0, ob0,
              idx1, uidx1, tidx1, rat1, urows1, trows1, ob1,
              sem_st, semg0, semg1, semw0, semw1):
    wid = lax.axis_index("s") * NC + lax.axis_index("c")
    slots = [
        dict(idx=idx0, uidx=uidx0, tidx=tidx0, rat=rat0, urows=urows0,
             trows=trows0, ob=ob0, semg=semg0, semw=semw0),
        dict(idx=idx1, uidx=uidx1, tidx=tidx1, rat=rat1, urows=urows1,
             trows=trows1, ob=ob1, semg=semg1, semw=semw1),
    ]
    pltpu.sync_copy(pe_hbm, pe_v)

    def base(c):
        return wid * ROWS_PER_W + c * CB

    def stage(c, sl):
        b0 = base(c)
        hs = [
            pltpu.async_copy(hist_hbm.at[pl.ds(b0 * TP, CB * TP)], sl["idx"],
                             sem_st),
            pltpu.async_copy(uid_hbm.at[pl.ds(b0, CB)], sl["uidx"], sem_st),
            pltpu.async_copy(tgt_hbm.at[pl.ds(b0, CB)], sl["tidx"], sem_st),
            pltpu.async_copy(rat_hbm.at[pl.ds(b0 * EMBED, CB * EMBED)],
                             sl["rat"], sem_st),
        ]
        for h in hs:
            h.wait()

    def gather_descs(sl):
        descs = []
        for b in range(CB):
            descs.append((tok_hbm.at[sl["idx"].at[pl.ds(b * TP, T)]],
                          sl["ob"].at[b, pl.ds(1, T)]))
        descs.append((usr_hbm.at[sl["uidx"]], sl["urows"]))
        descs.append((tok_hbm.at[sl["tidx"]], sl["trows"]))
        return descs

    def fire_gathers(sl):
        for src, dst in gather_descs(sl):
            pltpu.async_copy(src, dst, sl["semg"])

    def wait_gathers(sl):
        for src, dst in gather_descs(sl):
            pltpu.make_async_copy(src, dst, sl["semg"]).wait()

    def fire_write(c, sl):
        pltpu.async_copy(sl["ob"], out_hbm.at[pl.ds(base(c), CB)], sl["semw"])

    def wait_write(sl):
        pltpu.make_async_copy(sl["ob"], out_hbm.at[pl.ds(0, CB)],
                              sl["semw"]).wait()

    def compute(sl):
        rat_v, ob_v = sl["rat"], sl["ob"]
        urows_v, trows_v = sl["urows"], sl["trows"]

        def pair_body(bi, c):
            bs = [2 * bi, 2 * bi + 1]
            for tb in range((T + LANES - 1) // LANES):
                rvs = [rat_v[pl.ds(b * EMBED + tb * LANES, LANES)] for b in bs]
                for tt in range(min(LANES, T - tb * LANES)):
                    t = tb * LANES + tt
                    lane = jnp.full((LANES, 1), tt, dtype=jnp.int32)
                    pes = [pe_v[pl.ds(t * EMBED + e * LANES, LANES)]
                           for e in range(EV)]
                    for b, rv in zip(bs, rvs):
                        r = lax.gather(
                            rv, lane, _GATHER_DNUMS, (1,),
                            mode=lax.GatherScatterMode.PROMISE_IN_BOUNDS)
                        for e in range(EV):
                            sl_ = pl.ds(e * LANES, LANES)
                            ob_v[b, 1 + t, sl_] = (ob_v[b, 1 + t, sl_] + pes[e]) * r
            return c

        lax.fori_loop(0, CB // 2, pair_body, 0)

        for b in range(CB):
            for e in range(EV):
                sl_ = pl.ds(e * LANES, LANES)
                ob_v[b, 0, sl_] = urows_v[b, sl_]
                ob_v[b, S - 1, sl_] = trows_v[b, sl_]

    def phase(c, s, do_wait_write=True):
        o = 1 - s
        cn = jnp.minimum(c + 1, NCH - 1)
        stage(cn, slots[o])
        if do_wait_write:
            wait_write(slots[o])
        fire_gathers(slots[o])
        wait_gathers(slots[s])
        compute(slots[s])
        fire_write(c, slots[s])

    stage(0, slots[0])
    fire_gathers(slots[0])
    phase(0, 0, do_wait_write=False)
    phase(1, 1)

    def pair_loop(k, carry):
        phase(2 * k, 0)
        phase(2 * k + 1, 1)
        return carry

    lax.fori_loop(1, NCH // 2, pair_loop, 0)

    wait_gathers(slots[0])
    wait_write(slots[1])


def kernel(user_id, product_history, target_product_id, product_history_ratings,
           token_table, user_table):
    pe = jnp.asarray(_PE).reshape(-1)
    hist_p = jnp.pad(product_history, ((0, 0), (0, TP - T))).reshape(-1)
    rat_p = jnp.pad(product_history_ratings,
                    ((0, 0), (0, EMBED - T))).reshape(-1)
    return _sc_embed(user_id, hist_p, target_product_id, rat_p, pe,
                     token_table, user_table)


# speedup vs baseline: 1.8819x; 1.1975x over previous
import functools

import numpy as np
import jax
import jax.numpy as jnp
from jax import lax
from jax.experimental import pallas as pl
from jax.experimental.pallas import tpu as pltpu
from jax.experimental.pallas import tpu_sc as plsc

B, T = 16384, 50
EMBED = 64
S = T + 2
LANES = 16
EV = EMBED // LANES
TP = 56

NC, NS = 2, 16
NW = NC * NS
ROWS_PER_W = B // NW
CB = 16
NCH = ROWS_PER_W // CB


def _positional_encoding(max_len, d_model):
    pos = np.arange(max_len, dtype=np.float32)[:, None]
    div = np.exp(np.arange(0, d_model, 2, dtype=np.float32) * (-np.log(10000.0) / d_model))
    pe = np.zeros((max_len, d_model), dtype=np.float32)
    pe[:, 0::2] = np.sin(pos * div)
    pe[:, 1::2] = np.cos(pos * div)
    return pe


_PE = _positional_encoding(T, EMBED)

_mesh = plsc.VectorSubcoreMesh(core_axis_name="c", subcore_axis_name="s")

_GATHER_DNUMS = lax.GatherDimensionNumbers(
    offset_dims=(), collapsed_slice_dims=(0,), start_index_map=(0,))

_SLOT_SCRATCH = [
    pltpu.VMEM((CB * TP,), jnp.int32),
    pltpu.VMEM((CB,), jnp.int32),
    pltpu.VMEM((T * CB,), jnp.float32),
    pltpu.VMEM((CB, EMBED), jnp.float32),
    pltpu.VMEM((CB, S, EMBED), jnp.float32),
]


@functools.partial(
    pl.kernel,
    mesh=_mesh,
    compiler_params=pltpu.CompilerParams(use_tc_tiling_on_sc=False),
    out_type=jax.ShapeDtypeStruct((B, S, EMBED), jnp.float32),
    scratch_types=[
        pltpu.VMEM((T * EMBED,), jnp.float32),
        *_SLOT_SCRATCH,
        *_SLOT_SCRATCH,
        pltpu.SemaphoreType.DMA,
        pltpu.SemaphoreType.DMA,
        pltpu.SemaphoreType.DMA,
        pltpu.SemaphoreType.DMA,
        pltpu.SemaphoreType.DMA,
    ],
)
def _sc_embed(uid_hbm, hist_hbm, rat_hbm, pe_hbm, tok_hbm, usr_hbm,
              out_hbm, pe_v,
              idx0, uidx0, rat0, urows0, ob0,
              idx1, uidx1, rat1, urows1, ob1,
              sem_st, semg0, semg1, semw0, semw1):
    wid = lax.axis_index("s") * NC + lax.axis_index("c")
    slots = [
        dict(idx=idx0, uidx=uidx0, rat=rat0, urows=urows0, ob=ob0,
             semg=semg0, semw=semw0),
        dict(idx=idx1, uidx=uidx1, rat=rat1, urows=urows1, ob=ob1,
             semg=semg1, semw=semw1),
    ]
    pltpu.sync_copy(pe_hbm, pe_v)

    def base(c):
        return wid * ROWS_PER_W + c * CB

    def stage(c, sl):
        b0 = base(c)
        hs = [
            pltpu.async_copy(hist_hbm.at[pl.ds(b0 * TP, CB * TP)], sl["idx"],
                             sem_st),
            pltpu.async_copy(uid_hbm.at[pl.ds(b0, CB)], sl["uidx"], sem_st),
            pltpu.async_copy(
                rat_hbm.at[pl.ds((wid * NCH + c) * T * CB, T * CB)],
                sl["rat"], sem_st),
        ]
        for h in hs:
            h.wait()

    def gather_descs(sl):
        descs = []
        for b in range(CB):
            descs.append((tok_hbm.at[sl["idx"].at[pl.ds(b * TP, T + 1)]],
                          sl["ob"].at[b, pl.ds(1, T + 1)]))
        descs.append((usr_hbm.at[sl["uidx"]], sl["urows"]))
        return descs

    def fire_gathers(sl):
        for src, dst in gather_descs(sl):
            pltpu.async_copy(src, dst, sl["semg"])

    def wait_gathers(sl):
        for src, dst in gather_descs(sl):
            pltpu.make_async_copy(src, dst, sl["semg"]).wait()

    def fire_write(c, sl):
        pltpu.async_copy(sl["ob"], out_hbm.at[pl.ds(base(c), CB)], sl["semw"])

    def wait_write(sl):
        pltpu.make_async_copy(sl["ob"], out_hbm.at[pl.ds(0, CB)],
                              sl["semw"]).wait()

    def compute(sl):
        rat_v, ob_v = sl["rat"], sl["ob"]
        urows_v = sl["urows"]
        lanes = [jnp.full((LANES, 1), b, dtype=jnp.int32) for b in range(CB)]

        def t_body(t, c):
            rv = rat_v[pl.ds(t * CB, CB)]
            pes = [pe_v[pl.ds(t * EMBED + e * LANES, LANES)]
                   for e in range(EV)]
            for b in range(CB):
                r = lax.gather(
                    rv, lanes[b], _GATHER_DNUMS, (1,),
                    mode=lax.GatherScatterMode.PROMISE_IN_BOUNDS)
                for e in range(EV):
                    sl_ = pl.ds(e * LANES, LANES)
                    ob_v[b, 1 + t, sl_] = (ob_v[b, 1 + t, sl_] + pes[e]) * r
            return c

        lax.fori_loop(0, T, t_body, 0)

        for b in range(CB):
            for e in range(EV):
                sl_ = pl.ds(e * LANES, LANES)
                ob_v[b, 0, sl_] = urows_v[b, sl_]

    def phase(c, s, write_guard=None):
        o = 1 - s
        cn = jnp.minimum(c + 1, NCH - 1)
        stage(cn, slots[o])
        if write_guard is None:
            wait_write(slots[o])
        else:
            pl.when(write_guard)(lambda: wait_write(slots[o]))
        fire_gathers(slots[o])
        wait_gathers(slots[s])
        compute(slots[s])
        fire_write(c, slots[s])

    stage(0, slots[0])
    fire_gathers(slots[0])

    def pair_loop(k, carry):
        phase(2 * k, 0, write_guard=k > 0)
        phase(2 * k + 1, 1)
        return carry

    lax.fori_loop(0, NCH // 2, pair_loop, 0)

    wait_gathers(slots[0])
    wait_write(slots[1])


def kernel(user_id, product_history, target_product_id, product_history_ratings,
           token_table, user_table):
    pe = jnp.asarray(_PE).reshape(-1)
    hist_p = jnp.concatenate(
        [product_history, target_product_id[:, None],
         jnp.zeros((B, TP - T - 1), dtype=product_history.dtype)],
        axis=1).reshape(-1)
    rat_p = product_history_ratings.reshape(
        B // CB, CB, T).transpose(0, 2, 1).reshape(-1)
    return _sc_embed(user_id, hist_p, rat_p, pe, token_table, user_table)
